# baseline (speedup 1.0000x reference)
import functools

import jax
import jax.numpy as jnp
from jax import lax
from jax.experimental import pallas as pl
from jax.experimental.pallas import tpu as pltpu
from jax.experimental.pallas import tpu_sc as plsc

N = 100000
G = 512
D = 128
NC, NS, L = 2, 16, 16
W = NC * NS
BLK = 112
NBLK = 28
CHUNK = BLK * NBLK
NPAD = CHUNK * W
NB_FULL = N // BLK
TAIL_START = NB_FULL * BLK
GA = 640
SLICE = GA // NS
CR, CC = 16, 128
NT = 3

_mesh = plsc.VectorSubcoreMesh(
    core_axis_name="c", subcore_axis_name="s", num_cores=NC, num_subcores=NS)
_params = pltpu.CompilerParams(needs_layout_passes=False)


@functools.partial(
    pl.kernel,
    out_type=(
        jax.ShapeDtypeStruct((NC, GA, D), jnp.float32),
        jax.ShapeDtypeStruct((NC, CR, CC), jnp.float32),
    ),
    mesh=_mesh,
    compiler_params=_params,
    scratch_types=[
        pltpu.VMEM((2, BLK, D), jnp.float32),
        pltpu.VMEM((NT * NBLK, BLK), jnp.int32),
        pltpu.VMEM((GA, D), jnp.float32),
        pltpu.VMEM((CR, CC), jnp.float32),
        pltpu.VMEM((1, L), jnp.int32),
        pltpu.VMEM_SHARED((GA, D), jnp.float32),
        pltpu.VMEM_SHARED((CR, CC), jnp.float32),
        pltpu.SemaphoreType.DMA,
        pltpu.SemaphoreType.DMA,
    ],
)
def _phase1(na, nb, nm, ia, ib, im, ta, tb, tm,
            partials, counts, rowbuf, idsmat, acc_local, cnt, idbuf,
            acc_sh, cnt_sh, semi, semr):
    cid = lax.axis_index("c")
    sid = lax.axis_index("s")
    wid = cid * NS + sid
    zeros = jnp.zeros((L,), jnp.float32)
    ones = jnp.ones((L,), jnp.float32)
    types = ((na, ia, ta), (nb, ib, tb), (nm, im, tm))

    for t, (_, ids, _t) in enumerate(types):
        def _fire_ids(b, _, ids=ids, t=t):
            pltpu.async_copy(ids.at[pl.ds((wid * NBLK + b) * BLK, BLK)],
                             idsmat.at[t * NBLK + b], semi)
            return 0
        lax.fori_loop(0, NBLK, _fire_ids, 0)

    def _zfill(i, _):
        for j in range(D // L):
            acc_local[i, pl.ds(j * L, L)] = zeros
        return 0
    lax.fori_loop(0, GA, _zfill, 0)
    for r in range(CR):
        for j in range(CC // L):
            cnt[r, pl.ds(j * L, L)] = zeros
    idbuf[0, :] = lax.iota(jnp.int32, L)

    pltpu.sync_copy(acc_local.at[pl.ds(sid * SLICE, SLICE)],
                    acc_sh.at[pl.ds(sid * SLICE, SLICE)])

    @pl.when(sid == 0)
    def _():
        pltpu.sync_copy(acc_local.at[pl.ds(0, CR)], cnt_sh)
    plsc.subcore_barrier()

    def _drain_ids(b, _):
        pltpu.make_async_copy(ia.at[pl.ds(0, BLK)], idsmat.at[b], semi).wait()
        return 0
    lax.fori_loop(0, NT * NBLK, _drain_ids, 0)

    for t, (nodes, _ids, tail) in enumerate(types):
        def _issue_row(b, k, nodes=nodes, tail=tail):
            B = wid * NBLK + b

            @pl.when(B < NB_FULL)
            def _():
                pltpu.async_copy(nodes.at[pl.ds(B * BLK, BLK)],
                                 rowbuf.at[k], semr)

            @pl.when(B >= NB_FULL)
            def _():
                pltpu.async_copy(tail, rowbuf.at[k], semr)

        def _wait_row(k, nodes=nodes):
            pltpu.make_async_copy(nodes.at[pl.ds(0, BLK)], rowbuf.at[k],
                                  semr).wait()

        _issue_row(0, 0)
        _issue_row(1, 1)

        def _pair(g, _, t=t):
            for k in range(2):
                b = 2 * g + k
                _wait_row(k)
                irow = t * NBLK + b

                def _group(gr, _2, k=k, irow=irow):
                    idv = idsmat[irow, pl.ds(gr * L, L)]
                    plsc.addupdate_scatter(cnt, [idv >> 7, idv & 127], ones)
                    kv = jnp.full((L,), k, jnp.int32)
                    rowv = gr * L + lax.iota(jnp.int32, L)
                    for c in range(D):
                        cv = jnp.full((L,), c, jnp.int32)
                        v = plsc.load_gather(rowbuf, [kv, rowv, cv])
                        plsc.addupdate_scatter(acc_local, [idv, cv], v)
                    return 0
                lax.fori_loop(0, BLK // L, _group, 0)

                @pl.when(b + 2 < NBLK)
                def _():
                    _issue_row(b + 2, k)
            return 0
        lax.fori_loop(0, NBLK // 2, _pair, 0)

    lo = idsmat[0, pl.ds(0, L)][0]
    hi = idsmat[NBLK - 1, pl.ds(BLK - L, L)][L - 1]
    for t in range(1, NT):
        lo = jnp.minimum(lo, idsmat[t * NBLK, pl.ds(0, L)][0])
        hi = jnp.maximum(hi, idsmat[(t + 1) * NBLK - 1,
                                    pl.ds(BLK - L, L)][L - 1])
    lo16 = lo & ~(L - 1)
    ngran = (hi - lo16) // L + 1

    def _flush(g, _):
        base = lo16 + g * L
        idxv = base + lax.iota(jnp.int32, L)
        pltpu.sync_copy(acc_local.at[pl.ds(base, L)], acc_sh.at[idxv],
                        add=True)
        return 0
    lax.fori_loop(0, ngran, _flush, 0)

    pltpu.sync_copy(cnt, cnt_sh.at[idbuf.at[0]], add=True)
    plsc.subcore_barrier()

    pltpu.sync_copy(acc_sh.at[pl.ds(sid * SLICE, SLICE)],
                    partials.at[cid, pl.ds(sid * SLICE, SLICE)])

    @pl.when(sid < 2)
    def _():
        pltpu.sync_copy(cnt_sh.at[pl.ds(sid * (CR // 2), CR // 2)],
                        counts.at[cid, pl.ds(sid * (CR // 2), CR // 2)])


def _phase2_body(partials_ref, counts_ref, out_ref):
    s = partials_ref[0, :G, :] + partials_ref[1, :G, :]
    c = counts_ref[0, :4, :] + counts_ref[1, :4, :]
    cflat = c.reshape(G)
    inv = 1.0 / jnp.maximum(cflat, 1.0)
    out_ref[...] = s * inv[:, None]


def _phase2(partials, counts):
    return pl.pallas_call(
        _phase2_body,
        out_shape=jax.ShapeDtypeStruct((G, D), jnp.float32),
    )(partials, counts)


def kernel(nodes_atoms, nodes_bonds, nodes_monosacchs,
           batch_atoms, batch_bonds, batch_monosacchs):
    pad_ids = jnp.full((NPAD - N,), G, jnp.int32)
    ids = [jnp.concatenate([b, pad_ids])
           for b in (batch_atoms, batch_bonds, batch_monosacchs)]
    pad_rows = jnp.zeros((BLK - (N - TAIL_START), D), jnp.float32)
    tails = [jnp.concatenate([n[TAIL_START:N], pad_rows])
             for n in (nodes_atoms, nodes_bonds, nodes_monosacchs)]
    partials, counts = _phase1(nodes_atoms, nodes_bonds, nodes_monosacchs,
                               *ids, *tails)
    return _phase2(partials, counts)


# speedup vs baseline: 14.6347x; 14.6347x over previous
import functools

import jax
import jax.numpy as jnp
from jax import lax
from jax.experimental import pallas as pl
from jax.experimental.pallas import tpu as pltpu
from jax.experimental.pallas import tpu_sc as plsc

N = 100000
G = 512
D = 128
NC, NS, L = 2, 16, 16
W = NC * NS
BLK = 112
NBLK = 28
CHUNK = BLK * NBLK
NPAD = CHUNK * W
NB_FULL = N // BLK
TAIL_START = NB_FULL * BLK
GA = 640
SLICE = GA // NS
CR, CC = 16, 128
NT = 3
BANK = 2
NWAVE = NBLK // BANK

_mesh = plsc.VectorSubcoreMesh(
    core_axis_name="c", subcore_axis_name="s", num_cores=NC, num_subcores=NS)
_params = pltpu.CompilerParams(needs_layout_passes=False)


@functools.partial(
    pl.kernel,
    out_type=(
        jax.ShapeDtypeStruct((NC, GA, D), jnp.float32),
        jax.ShapeDtypeStruct((NC, CR, CC), jnp.float32),
    ),
    mesh=_mesh,
    compiler_params=_params,
    scratch_types=[
        pltpu.VMEM((2 * BANK, BLK, D), jnp.float32),
        pltpu.VMEM((NT * NBLK, BLK), jnp.int32),
        pltpu.VMEM((SLICE, D), jnp.float32),
        pltpu.VMEM((CR, CC), jnp.float32),
        pltpu.VMEM((1, L), jnp.int32),
        pltpu.VMEM_SHARED((GA, D), jnp.float32),
        pltpu.VMEM_SHARED((CR, CC), jnp.float32),
        pltpu.SemaphoreType.DMA,
        pltpu.SemaphoreType.DMA,
        pltpu.SemaphoreType.DMA,
    ],
)
def _phase1(na, nb, nm, ia, ib, im, ta, tb, tm,
            partials, counts, rowbuf, idsmat, zbuf, cnt, idbuf,
            acc_sh, cnt_sh, semi, semr, sems):
    cid = lax.axis_index("c")
    sid = lax.axis_index("s")
    wid = cid * NS + sid
    zeros = jnp.zeros((L,), jnp.float32)
    ones = jnp.ones((L,), jnp.float32)
    types = ((na, ia, ta), (nb, ib, tb), (nm, im, tm))

    for t, (_, ids, _t) in enumerate(types):
        def _fire_ids(b, _, ids=ids, t=t):
            pltpu.async_copy(ids.at[pl.ds((wid * NBLK + b) * BLK, BLK)],
                             idsmat.at[t * NBLK + b], semi)
            return 0
        lax.fori_loop(0, NBLK, _fire_ids, 0)

    for k0 in range(BANK):
        B0 = wid * NBLK + k0

        @pl.when(B0 < NB_FULL)
        def _(k0=k0, B0=B0):
            pltpu.async_copy(na.at[pl.ds(B0 * BLK, BLK)], rowbuf.at[k0], semr)

        @pl.when(B0 >= NB_FULL)
        def _(k0=k0):
            pltpu.async_copy(ta, rowbuf.at[k0], semr)

    def _zfill(i, _):
        for j in range(D // L):
            zbuf[i, pl.ds(j * L, L)] = zeros
        return 0
    lax.fori_loop(0, SLICE, _zfill, 0)
    for r in range(CR):
        for j in range(CC // L):
            cnt[r, pl.ds(j * L, L)] = zeros
    idbuf[0, :] = lax.iota(jnp.int32, L)

    pltpu.sync_copy(zbuf, acc_sh.at[pl.ds(sid * SLICE, SLICE)])

    @pl.when(sid == 0)
    def _():
        pltpu.sync_copy(zbuf.at[pl.ds(0, CR)], cnt_sh)
    plsc.subcore_barrier()

    def _drain_ids(b, _):
        pltpu.make_async_copy(ia.at[pl.ds(0, BLK)], idsmat.at[b], semi).wait()
        return 0
    lax.fori_loop(0, NT * NBLK, _drain_ids, 0)

    for t, (nodes, _ids, tail) in enumerate(types):
        def _issue_row(b, k, nodes=nodes, tail=tail):
            B = wid * NBLK + b

            @pl.when(B < NB_FULL)
            def _():
                pltpu.async_copy(nodes.at[pl.ds(B * BLK, BLK)],
                                 rowbuf.at[k], semr)

            @pl.when(B >= NB_FULL)
            def _():
                pltpu.async_copy(tail, rowbuf.at[k], semr)

        def _wait_row(k, nodes=nodes):
            pltpu.make_async_copy(nodes.at[pl.ds(0, BLK)], rowbuf.at[k],
                                  semr).wait()

        def _wait_scatter(k, t=t):
            pltpu.make_async_copy(
                rowbuf.at[k], acc_sh.at[idsmat.at[t * NBLK]], sems).wait()

        if t > 0:
            for k in range(BANK):
                _issue_row(k, k)

        def _wave(g, _, t=t):
            bank = g % 2

            @pl.when(g > 0)
            def _():
                for k in range(BANK):
                    _wait_scatter((1 - bank) * BANK + k)

            @pl.when(g + 1 < NWAVE)
            def _():
                for k in range(BANK):
                    _issue_row(2 * (g + 1) + k, (1 - bank) * BANK + k)

            for k in range(BANK):
                slot = bank * BANK + k
                _wait_row(slot)
                pltpu.async_copy(
                    rowbuf.at[slot],
                    acc_sh.at[idsmat.at[t * NBLK + 2 * g + k]], sems,
                    add=True)
            return 0
        lax.fori_loop(0, NWAVE, _wave, 0)

        def _count(r, _, t=t):
            for j in range(BLK // L):
                idv = idsmat[t * NBLK + r, pl.ds(j * L, L)]
                plsc.addupdate_scatter(cnt, [idv >> 7, idv & 127], ones)
            return 0
        lax.fori_loop(0, NBLK, _count, 0)

        for k in range(BANK):
            _wait_scatter(((NWAVE - 1) % 2) * BANK + k)

    pltpu.sync_copy(cnt, cnt_sh.at[idbuf.at[0]], add=True)
    plsc.subcore_barrier()

    pltpu.sync_copy(acc_sh.at[pl.ds(sid * SLICE, SLICE)],
                    partials.at[cid, pl.ds(sid * SLICE, SLICE)])

    @pl.when(sid < 2)
    def _():
        pltpu.sync_copy(cnt_sh.at[pl.ds(sid * (CR // 2), CR // 2)],
                        counts.at[cid, pl.ds(sid * (CR // 2), CR // 2)])


def _phase2_body(partials_ref, counts_ref, out_ref):
    s = partials_ref[0, :G, :] + partials_ref[1, :G, :]
    c = counts_ref[0, :4, :] + counts_ref[1, :4, :]
    cflat = c.reshape(G)
    inv = 1.0 / jnp.maximum(cflat, 1.0)
    out_ref[...] = s * inv[:, None]


def _phase2(partials, counts):
    return pl.pallas_call(
        _phase2_body,
        out_shape=jax.ShapeDtypeStruct((G, D), jnp.float32),
    )(partials, counts)


def kernel(nodes_atoms, nodes_bonds, nodes_monosacchs,
           batch_atoms, batch_bonds, batch_monosacchs):
    pad_ids = jnp.full((NPAD - N,), G, jnp.int32)
    ids = [jnp.concatenate([b, pad_ids])
           for b in (batch_atoms, batch_bonds, batch_monosacchs)]
    pad_rows = jnp.zeros((BLK - (N - TAIL_START), D), jnp.float32)
    tails = [jnp.concatenate([n[TAIL_START:N], pad_rows])
             for n in (nodes_atoms, nodes_bonds, nodes_monosacchs)]
    partials, counts = _phase1(nodes_atoms, nodes_bonds, nodes_monosacchs,
                               *ids, *tails)
    return _phase2(partials, counts)
